# Initial kernel scaffold; baseline (speedup 1.0000x reference)
import jax
import jax.numpy as jnp
from jax.experimental import pallas as pl


def kernel(edge_index, edge_type, r_to_e, r_rel, dynamic_emb, emb_rel, gru_w_ih, gru_w_hh, gru_b_ih, gru_b_hh, time_w, time_b, wn0, wl0, we0, wn1, wl1, we1):
    raise NotImplementedError("write your pallas kernel here")


# speedup vs baseline: 3.9515x; 3.9515x over previous
import functools

import jax
import jax.numpy as jnp
from jax import lax
from jax.experimental import pallas as pl
from jax.experimental.pallas import tpu as pltpu
from jax.experimental.pallas import tpu_sc as plsc

N_ENTS = 10000
H = 128
R2 = 400
NE = 320000

NC = 2
NS = 16
NW = NC * NS
EW = NE // NW
CH = 80
NCH = EW // CH

NP = 10240
RP = 512
ROWS_W = NP // NS
RROWS_W = RP // NS

_SLOPE = (1.0 / 8.0 + 1.0 / 3.0) / 2.0


def _wid():
    c = lax.axis_index("c")
    s = lax.axis_index("s")
    return c, s, c * NS + s


def _sc_pool_body(h_hbm, rte_hbm, rrel_hbm, dst_hbm, ones_hbm,
                  zpool_hbm, zcnt_hbm, zdeg_hbm,
                  pool_out, cnt_out, deg_out,
                  idx_e, idx_r, idx_d, rows_v, ones_v,
                  pool_sh, cnt_sh, deg_sh, sem):
    c, s, wid = _wid()
    rr0 = s * RROWS_W
    r0 = s * ROWS_W
    pltpu.sync_copy(zpool_hbm.at[pl.ds(rr0, RROWS_W)],
                    pool_sh.at[pl.ds(rr0, RROWS_W)])
    pltpu.sync_copy(zcnt_hbm.at[pl.ds(rr0, RROWS_W)],
                    cnt_sh.at[pl.ds(rr0, RROWS_W)])
    pltpu.sync_copy(zdeg_hbm.at[pl.ds(r0, ROWS_W)],
                    deg_sh.at[pl.ds(r0, ROWS_W)])
    pltpu.sync_copy(ones_hbm, ones_v)
    plsc.subcore_barrier()
    base = wid * EW

    def chunk(i, carry):
        off = base + i * CH
        pltpu.sync_copy(rte_hbm.at[pl.ds(off, CH)], idx_e)
        pltpu.async_copy(h_hbm.at[idx_e], rows_v, sem).wait()
        pltpu.sync_copy(rrel_hbm.at[pl.ds(off, CH)], idx_r)
        pltpu.sync_copy(dst_hbm.at[pl.ds(off, CH)], idx_d)
        pltpu.sync_copy(rows_v, pool_sh.at[idx_r], add=True)
        pltpu.sync_copy(ones_v, cnt_sh.at[idx_r], add=True)
        pltpu.sync_copy(ones_v, deg_sh.at[idx_d], add=True)
        return carry

    lax.fori_loop(0, NCH, chunk, 0)
    plsc.subcore_barrier()
    pltpu.sync_copy(pool_sh.at[pl.ds(rr0, RROWS_W)],
                    pool_out.at[c, pl.ds(rr0, RROWS_W)])
    pltpu.sync_copy(cnt_sh.at[pl.ds(rr0, RROWS_W)],
                    cnt_out.at[c, pl.ds(rr0, RROWS_W)])
    pltpu.sync_copy(deg_sh.at[pl.ds(r0, ROWS_W)],
                    deg_out.at[c, pl.ds(r0, ROWS_W)])


@functools.lru_cache(maxsize=None)
def _get_sc_pool():
    return pl.kernel(
        _sc_pool_body,
        out_type=(jax.ShapeDtypeStruct((NC, RP, H), jnp.float32),
                  jax.ShapeDtypeStruct((NC, RP, 16), jnp.float32),
                  jax.ShapeDtypeStruct((NC, NP, 16), jnp.float32)),
        mesh=plsc.VectorSubcoreMesh(core_axis_name="c", subcore_axis_name="s",
                                    num_cores=NC, num_subcores=NS),
        scratch_types=[
            pltpu.VMEM((CH,), jnp.int32),
            pltpu.VMEM((CH,), jnp.int32),
            pltpu.VMEM((CH,), jnp.int32),
            pltpu.VMEM((CH, H), jnp.float32),
            pltpu.VMEM((CH, 16), jnp.float32),
            pltpu.VMEM_SHARED((RP, H), jnp.float32),
            pltpu.VMEM_SHARED((RP, 16), jnp.float32),
            pltpu.VMEM_SHARED((NP, 16), jnp.float32),
            pltpu.SemaphoreType.DMA,
        ],
    )


def _sc_pool(*args):
    return _get_sc_pool()(*args)


def _sc_layer_a_body(h_hbm, h0_hbm, src_hbm, et_hbm, dst_hbm, zacc_hbm,
                     rel_out, agg1_out,
                     idx_g, idx_d, rows_v, acc_sh, sem):
    c, s, wid = _wid()
    r0 = s * ROWS_W
    pltpu.sync_copy(zacc_hbm.at[pl.ds(r0, ROWS_W)],
                    acc_sh.at[pl.ds(r0, ROWS_W)])
    plsc.subcore_barrier()
    base = wid * EW

    def chunk_rel(i, carry):
        off = base + i * CH
        pltpu.sync_copy(et_hbm.at[pl.ds(off, CH)], idx_g)
        pltpu.async_copy(h0_hbm.at[idx_g], rows_v, sem).wait()
        pltpu.sync_copy(dst_hbm.at[pl.ds(off, CH)], idx_d)
        pltpu.sync_copy(rows_v, acc_sh.at[idx_d], add=True)
        return carry

    lax.fori_loop(0, NCH, chunk_rel, 0)
    plsc.subcore_barrier()
    pltpu.sync_copy(acc_sh.at[pl.ds(r0, ROWS_W)],
                    rel_out.at[c, pl.ds(r0, ROWS_W)])
    plsc.subcore_barrier()

    def chunk_h(i, carry):
        off = base + i * CH
        pltpu.sync_copy(src_hbm.at[pl.ds(off, CH)], idx_g)
        pltpu.async_copy(h_hbm.at[idx_g], rows_v, sem).wait()
        pltpu.sync_copy(dst_hbm.at[pl.ds(off, CH)], idx_d)
        pltpu.sync_copy(rows_v, acc_sh.at[idx_d], add=True)
        return carry

    lax.fori_loop(0, NCH, chunk_h, 0)
    plsc.subcore_barrier()
    pltpu.sync_copy(acc_sh.at[pl.ds(r0, ROWS_W)],
                    agg1_out.at[c, pl.ds(r0, ROWS_W)])


@functools.lru_cache(maxsize=None)
def _get_sc_layer_a():
    return pl.kernel(
        _sc_layer_a_body,
        out_type=(jax.ShapeDtypeStruct((NC, NP, H), jnp.float32),
                  jax.ShapeDtypeStruct((NC, NP, H), jnp.float32)),
        mesh=plsc.VectorSubcoreMesh(core_axis_name="c", subcore_axis_name="s",
                                    num_cores=NC, num_subcores=NS),
        scratch_types=[
            pltpu.VMEM((CH,), jnp.int32),
            pltpu.VMEM((CH,), jnp.int32),
            pltpu.VMEM((CH, H), jnp.float32),
            pltpu.VMEM_SHARED((NP, H), jnp.float32),
            pltpu.SemaphoreType.DMA,
        ],
    )


def _sc_layer_a(*args):
    return _get_sc_layer_a()(*args)


def _sc_layer_b_body(cur_hbm, src_hbm, dst_hbm, rel_hbm,
                     agg2_out,
                     idx_g, idx_d, rows_v, acc_sh, sem):
    c, s, wid = _wid()
    r0 = s * ROWS_W
    pltpu.sync_copy(rel_hbm.at[c, pl.ds(r0, ROWS_W)],
                    acc_sh.at[pl.ds(r0, ROWS_W)])
    plsc.subcore_barrier()
    base = wid * EW

    def chunk(i, carry):
        off = base + i * CH
        pltpu.sync_copy(src_hbm.at[pl.ds(off, CH)], idx_g)
        pltpu.async_copy(cur_hbm.at[idx_g], rows_v, sem).wait()
        pltpu.sync_copy(dst_hbm.at[pl.ds(off, CH)], idx_d)
        pltpu.sync_copy(rows_v, acc_sh.at[idx_d], add=True)
        return carry

    lax.fori_loop(0, NCH, chunk, 0)
    plsc.subcore_barrier()
    pltpu.sync_copy(acc_sh.at[pl.ds(r0, ROWS_W)],
                    agg2_out.at[c, pl.ds(r0, ROWS_W)])


@functools.lru_cache(maxsize=None)
def _get_sc_layer_b():
    return pl.kernel(
        _sc_layer_b_body,
        out_type=jax.ShapeDtypeStruct((NC, NP, H), jnp.float32),
        mesh=plsc.VectorSubcoreMesh(core_axis_name="c", subcore_axis_name="s",
                                    num_cores=NC, num_subcores=NS),
        scratch_types=[
            pltpu.VMEM((CH,), jnp.int32),
            pltpu.VMEM((CH,), jnp.int32),
            pltpu.VMEM((CH, H), jnp.float32),
            pltpu.VMEM_SHARED((NP, H), jnp.float32),
            pltpu.SemaphoreType.DMA,
        ],
    )


def _sc_layer_b(*args):
    return _get_sc_layer_b()(*args)


def _l2_body(x_ref, o_ref):
    x = x_ref[...]
    n = jnp.sqrt(jnp.sum(x * x, axis=-1, keepdims=True))
    o_ref[...] = x / jnp.maximum(n, 1e-12)


def _tc_l2(x):
    nb = 8
    rb = x.shape[0] // nb
    return pl.pallas_call(
        _l2_body,
        grid=(nb,),
        in_specs=[pl.BlockSpec((rb, H), lambda i: (i, 0))],
        out_specs=pl.BlockSpec((rb, H), lambda i: (i, 0)),
        out_shape=jax.ShapeDtypeStruct(x.shape, jnp.float32),
    )(x)


def _dot_t(a, b):
    return lax.dot_general(a, b, (((1,), (1,)), ((), ())),
                           preferred_element_type=jnp.float32)


def _gru_body(er_ref, pool_ref, cnt_ref, wih_ref, whh_ref, bih_ref, bhh_ref,
              h0_ref):
    er = er_ref[...]
    sums = pool_ref[0, :R2, :] + pool_ref[1, :R2, :]
    cnts = cnt_ref[0, :R2, 0:1] + cnt_ref[1, :R2, 0:1]
    x_mean = sums / jnp.maximum(cnts, 1.0)
    wih = wih_ref[...]
    gi = (_dot_t(er, wih[:, :H]) + _dot_t(x_mean, wih[:, H:])
          + bih_ref[...])
    gh = _dot_t(er, whh_ref[...]) + bhh_ref[...]
    r = jax.nn.sigmoid(gi[:, :H] + gh[:, :H])
    z = jax.nn.sigmoid(gi[:, H:2 * H] + gh[:, H:2 * H])
    n = jnp.tanh(gi[:, 2 * H:] + r * gh[:, 2 * H:])
    h0 = (1.0 - z) * n + z * er
    nn = jnp.sqrt(jnp.sum(h0 * h0, axis=-1, keepdims=True))
    h0_ref[...] = h0 / jnp.maximum(nn, 1e-12)


def _tc_gru(emb_rel, pool, cnt, w_ih, w_hh, b_ih, b_hh):
    return pl.pallas_call(
        _gru_body,
        out_shape=jax.ShapeDtypeStruct((R2, H), jnp.float32),
    )(emb_rel, pool, cnt, w_ih, w_hh, b_ih, b_hh)


def _layer_body(agg_ref, deg_ref, cur_ref, wn_ref, wl_ref, we_ref, o_ref):
    a = agg_ref[0] + agg_ref[1]
    deg = deg_ref[0, :, 0:1] + deg_ref[1, :, 0:1]
    norm = 1.0 / jnp.maximum(deg, 1.0)
    cur = cur_ref[...]
    agg = jnp.dot(a, wn_ref[...], preferred_element_type=jnp.float32) * norm
    loop = jnp.where(deg > 0,
                     jnp.dot(cur, wl_ref[...],
                             preferred_element_type=jnp.float32),
                     jnp.dot(cur, we_ref[...],
                             preferred_element_type=jnp.float32))
    x = agg + loop
    o_ref[...] = jnp.where(x >= 0, x, x * _SLOPE)


def _tc_layer(agg, deg, cur, wn, wl, we):
    nb = 8
    rb = NP // nb
    wspec = pl.BlockSpec((H, H), lambda i: (0, 0))
    return pl.pallas_call(
        _layer_body,
        grid=(nb,),
        in_specs=[
            pl.BlockSpec((NC, rb, H), lambda i: (0, i, 0)),
            pl.BlockSpec((NC, rb, 16), lambda i: (0, i, 0)),
            pl.BlockSpec((rb, H), lambda i: (i, 0)),
            wspec, wspec, wspec,
        ],
        out_specs=pl.BlockSpec((rb, H), lambda i: (i, 0)),
        out_shape=jax.ShapeDtypeStruct((NP, H), jnp.float32),
    )(agg, deg, cur, wn, wl, we)


def _final_body(agg_ref, deg_ref, cur_ref, h_ref, wn_ref, wl_ref, we_ref,
                tw_ref, tb_ref, o_ref):
    a = agg_ref[0] + agg_ref[1]
    deg = deg_ref[0, :, 0:1] + deg_ref[1, :, 0:1]
    norm = 1.0 / jnp.maximum(deg, 1.0)
    cur = cur_ref[...]
    agg = jnp.dot(a, wn_ref[...], preferred_element_type=jnp.float32) * norm
    loop = jnp.where(deg > 0,
                     jnp.dot(cur, wl_ref[...],
                             preferred_element_type=jnp.float32),
                     jnp.dot(cur, we_ref[...],
                             preferred_element_type=jnp.float32))
    x = agg + loop
    cur2 = jnp.where(x >= 0, x, x * _SLOPE)
    nn = jnp.sqrt(jnp.sum(cur2 * cur2, axis=-1, keepdims=True))
    cur2 = cur2 / jnp.maximum(nn, 1e-12)
    h = h_ref[...]
    tw = jax.nn.sigmoid(jnp.dot(h, tw_ref[...],
                                preferred_element_type=jnp.float32)
                        + tb_ref[...])
    o_ref[...] = tw * cur2 + (1.0 - tw) * h


def _tc_final(agg, deg, cur1, h, wn, wl, we, time_w, time_b):
    nb = 8
    rb = NP // nb
    wspec = pl.BlockSpec((H, H), lambda i: (0, 0))
    return pl.pallas_call(
        _final_body,
        grid=(nb,),
        in_specs=[
            pl.BlockSpec((NC, rb, H), lambda i: (0, i, 0)),
            pl.BlockSpec((NC, rb, 16), lambda i: (0, i, 0)),
            pl.BlockSpec((rb, H), lambda i: (i, 0)),
            pl.BlockSpec((rb, H), lambda i: (i, 0)),
            wspec, wspec, wspec, wspec,
            pl.BlockSpec((1, H), lambda i: (0, 0)),
        ],
        out_specs=pl.BlockSpec((rb, H), lambda i: (i, 0)),
        out_shape=jax.ShapeDtypeStruct((NP, H), jnp.float32),
    )(agg, deg, cur1, h, wn, wl, we, time_w, time_b)


def kernel(edge_index, edge_type, r_to_e, r_rel, dynamic_emb, emb_rel,
           gru_w_ih, gru_w_hh, gru_b_ih, gru_b_hh, time_w, time_b,
           wn0, wl0, we0, wn1, wl1, we1):
    f32 = jnp.float32
    src = edge_index[0].astype(jnp.int32)
    dst = edge_index[1].astype(jnp.int32)
    et = edge_type.astype(jnp.int32)
    rte = r_to_e.astype(jnp.int32)
    rrel = r_rel.astype(jnp.int32)

    emb_pad = jnp.zeros((NP, H), f32).at[:N_ENTS].set(dynamic_emb)
    ones = jnp.ones((CH, 16), f32)
    zpool = jnp.zeros((RP, H), f32)
    zcnt = jnp.zeros((RP, 16), f32)
    zdeg = jnp.zeros((NP, 16), f32)
    zacc = jnp.zeros((NP, H), f32)

    h = _tc_l2(emb_pad)
    pool, cnt, deg = _sc_pool(h, rte, rrel, dst, ones, zpool, zcnt, zdeg)
    h0 = _tc_gru(emb_rel, pool, cnt, gru_w_ih, gru_w_hh,
                 gru_b_ih.reshape(1, -1), gru_b_hh.reshape(1, -1))
    rel, agg1 = _sc_layer_a(h, h0, src, et, dst, zacc)
    cur1 = _tc_layer(agg1, deg, h, wn0, wl0, we0)
    agg2 = _sc_layer_b(cur1, src, dst, rel)
    out = _tc_final(agg2, deg, cur1, h, wn1, wl1, we1,
                    time_w, time_b.reshape(1, -1))
    return out[:N_ENTS]
